# baseline (speedup 1.0000x reference)
import functools

import jax
import jax.numpy as jnp
from jax.experimental import pallas as pl
from jax.experimental.pallas import tpu as pltpu


def _pipeline_kernel(emb_hbm, out_hbm, in_buf, out_buf, in_sems, out_sems, *,
                     scale, chunk_rows, chunks):
    n = len(chunks)

    def start_in(i):
        base, rows = chunks[i]
        slot = i % 2
        pltpu.make_async_copy(
            emb_hbm.at[pl.ds(base, rows)],
            in_buf.at[slot, pl.ds(0, rows)],
            in_sems.at[slot],
        ).start()

    def wait_in(i):
        base, rows = chunks[i]
        slot = i % 2
        pltpu.make_async_copy(
            emb_hbm.at[pl.ds(base, rows)],
            in_buf.at[slot, pl.ds(0, rows)],
            in_sems.at[slot],
        ).wait()

    def start_out(i):
        base, rows = chunks[i]
        slot = i % 2
        pltpu.make_async_copy(
            out_buf.at[slot, pl.ds(0, rows)],
            out_hbm.at[pl.ds(base, rows)],
            out_sems.at[slot],
        ).start()

    def wait_out(i):
        base, rows = chunks[i]
        slot = i % 2
        pltpu.make_async_copy(
            out_buf.at[slot, pl.ds(0, rows)],
            out_hbm.at[pl.ds(base, rows)],
            out_sems.at[slot],
        ).wait()

    start_in(0)
    for i in range(n):
        if i + 1 < n:
            start_in(i + 1)
        wait_in(i)
        if i >= 2:
            wait_out(i - 2)
        rows = chunks[i][1]
        slot = i % 2
        out_buf[slot, pl.ds(0, rows)] = (
            in_buf[slot, pl.ds(0, rows)] * scale
        ).astype(out_buf.dtype)
        start_out(i)
    if n >= 2:
        wait_out(n - 2)
    wait_out(n - 1)


def kernel(x, embedding):
    max_seq_len, dim = embedding.shape
    seq_len = x.shape[1]
    if seq_len > max_seq_len:
        raise ValueError(f"seq_len={seq_len} exceeds max_seq_len={max_seq_len}")
    dtype = embedding.dtype
    itemsize = jnp.dtype(dtype).itemsize
    sub = max(8, 32 // itemsize)
    row_bytes = dim * itemsize

    target_chunk_bytes = 1 * 1024 * 1024
    chunk_rows = max(sub, min(seq_len, target_chunk_bytes // max(1, row_bytes)))
    chunk_rows = max(sub, (chunk_rows // sub) * sub)
    chunks = []
    base = 0
    while base < seq_len:
        rows = min(chunk_rows, seq_len - base)
        chunks.append((base, rows))
        base += rows

    vmem_bytes = 4 * chunk_rows * row_bytes
    vmem_limit = int(min(96 * 1024 * 1024,
                         max(16 * 1024 * 1024, vmem_bytes + 2 * 1024 * 1024)))

    return pl.pallas_call(
        functools.partial(_pipeline_kernel, scale=float(dim) ** -0.5,
                          chunk_rows=chunk_rows, chunks=chunks),
        out_shape=jax.ShapeDtypeStruct((seq_len, dim), dtype),
        in_specs=[pl.BlockSpec(memory_space=pl.ANY)],
        out_specs=pl.BlockSpec(memory_space=pl.ANY),
        scratch_shapes=[
            pltpu.VMEM((2, chunk_rows, dim), dtype),
            pltpu.VMEM((2, chunk_rows, dim), dtype),
            pltpu.SemaphoreType.DMA((2,)),
            pltpu.SemaphoreType.DMA((2,)),
        ],
        compiler_params=pltpu.CompilerParams(
            vmem_limit_bytes=vmem_limit,
        ),
    )(embedding)


# speedup vs baseline: 1.1334x; 1.1334x over previous
import functools

import jax
import jax.numpy as jnp
from jax.experimental import pallas as pl
from jax.experimental.pallas import tpu as pltpu


def _round_up(x, m):
    return ((x + m - 1) // m) * m


def _scale_kernel(emb_ref, out_ref, *, scale):
    out_ref[...] = (emb_ref[...] * scale).astype(out_ref.dtype)


def kernel(x, embedding):
    max_seq_len, dim = embedding.shape
    seq_len = x.shape[1]
    if seq_len > max_seq_len:
        raise ValueError(f"seq_len={seq_len} exceeds max_seq_len={max_seq_len}")
    dtype = embedding.dtype
    itemsize = jnp.dtype(dtype).itemsize
    sub = max(8, 32 // itemsize)
    row_bytes = dim * itemsize

    target_block_bytes = 2 * 1024 * 1024
    rows_budget = max(sub, target_block_bytes // max(1, row_bytes))
    block_rows = min(rows_budget, _round_up(seq_len, sub))
    block_rows = max(sub, (block_rows // sub) * sub)
    num_blocks = pl.cdiv(seq_len, block_rows)

    block_bytes = block_rows * row_bytes
    vmem_limit = int(min(96 * 1024 * 1024,
                         max(16 * 1024 * 1024, 6 * block_bytes)))

    return pl.pallas_call(
        functools.partial(_scale_kernel, scale=float(dim) ** -0.5),
        out_shape=jax.ShapeDtypeStruct((seq_len, dim), dtype),
        grid=(num_blocks,),
        in_specs=[pl.BlockSpec((block_rows, dim), lambda i: (i, 0))],
        out_specs=pl.BlockSpec((block_rows, dim), lambda i: (i, 0)),
        compiler_params=pltpu.CompilerParams(
            dimension_semantics=("arbitrary",),
            vmem_limit_bytes=vmem_limit,
        ),
    )(embedding)
